# baseline (speedup 1.0000x reference)
import functools

import jax
import jax.numpy as jnp
from jax import lax
from jax.experimental import pallas as pl
from jax.experimental.pallas import tpu as pltpu
from jax.experimental.pallas import tpu_sc as plsc

B = 1024
NGRAM = 10
Z = 10
D = 128
V = 100000
LANES = 16
NW = 32
ROWS_PER_W = D // NW
NCHUNK = B // LANES
DN = B + Z * B
SQ_ITERS = V // LANES

@functools.cache
def _sc_gather_built():
    mesh = plsc.VectorSubcoreMesh(core_axis_name="c", subcore_axis_name="s")
    return functools.partial(
        pl.kernel,
        out_type=[
            jax.ShapeDtypeStruct((D, B), jnp.float32),
            jax.ShapeDtypeStruct((D, B), jnp.float32),
            jax.ShapeDtypeStruct((D, Z * B), jnp.float32),
            jax.ShapeDtypeStruct((NW, 128), jnp.float32),
        ],
        mesh=mesh,
        compiler_params=pltpu.CompilerParams(needs_layout_passes=False),
        scratch_types=[
            pltpu.VMEM((V,), jnp.float32),
            pltpu.VMEM((DN,), jnp.int32),
            pltpu.VMEM((DN,), jnp.float32),
        ],
    )(_sc_gather_body)


def _sc_gather_body(rv_hbm, rd_hbm, widx_hbm, dnidx_hbm,
                    wp_hbm, docs_hbm, neg_hbm, reg_hbm,
                    row_v, idx_v, out_v):
    wid = lax.axis_index("s") * 2 + lax.axis_index("c")

    pltpu.sync_copy(widx_hbm, idx_v.at[pl.ds(0, NGRAM * B)])

    def rv_row(r, carry):
        d = wid * ROWS_PER_W + r
        pltpu.sync_copy(rv_hbm.at[d], row_v)

        def chunk(c, carry2):
            acc = jnp.zeros((LANES,), jnp.float32)
            for zi in range(NGRAM):
                iv = idx_v[pl.ds(zi * B + c * LANES, LANES)]
                acc = acc + plsc.load_gather(row_v, [iv])
            out_v[pl.ds(c * LANES, LANES)] = acc
            return carry2

        lax.fori_loop(0, NCHUNK, chunk, 0)
        pltpu.sync_copy(out_v.at[pl.ds(0, B)], wp_hbm.at[d])
        return carry

    lax.fori_loop(0, ROWS_PER_W, rv_row, 0)

    pltpu.sync_copy(dnidx_hbm, idx_v)

    def rd_row(r, regacc):
        d = wid * ROWS_PER_W + r
        pltpu.sync_copy(rd_hbm.at[d], row_v)

        def sq(i, a):
            v = row_v[pl.ds(i * LANES, LANES)]
            return a + v * v

        regacc = lax.fori_loop(0, SQ_ITERS, sq, regacc)

        def chunk(c, carry2):
            iv = idx_v[pl.ds(c * LANES, LANES)]
            out_v[pl.ds(c * LANES, LANES)] = plsc.load_gather(row_v, [iv])
            return carry2

        lax.fori_loop(0, DN // LANES, chunk, 0)
        pltpu.sync_copy(out_v.at[pl.ds(0, B)], docs_hbm.at[d])
        pltpu.sync_copy(out_v.at[pl.ds(B, Z * B)], neg_hbm.at[d])
        return regacc

    regacc = lax.fori_loop(0, ROWS_PER_W, rd_row, jnp.zeros((LANES,), jnp.float32))

    out_v[pl.ds(0, LANES)] = regacc
    for j in range(1, 128 // LANES):
        out_v[pl.ds(j * LANES, LANES)] = jnp.zeros((LANES,), jnp.float32)
    pltpu.sync_copy(out_v.at[pl.ds(0, 128)], reg_hbm.at[wid])


def _tc_finish_body(wp_ref, docs_ref, neg_ref, regp_ref, proj_ref, beta_ref,
                    out_ref):
    wpT = wp_ref[...] * (1.0 / NGRAM)
    norm = jnp.sqrt(jnp.sum(wpT * wpT, axis=0, keepdims=True))
    normedT = wpT / norm
    tpreT = jnp.dot(proj_ref[...], normedT,
                    preferred_element_type=jnp.float32)
    mean = jnp.mean(tpreT, axis=1, keepdims=True)
    std = jnp.sqrt(jnp.sum((tpreT - mean) ** 2, axis=1, keepdims=True)
                   / (B - 1))
    tT = jnp.clip((tpreT - mean) / jnp.sqrt(std) + beta_ref[...], -1.0, 1.0)

    pos_logit = jnp.sum(tT * docs_ref[...], axis=0, keepdims=True)
    p_pos = jnp.clip(1.0 / (1.0 + jnp.exp(-pos_logit)), -0.999, 0.999)
    log_p = float(Z) * jnp.log(p_pos)

    nsample_log = jnp.zeros((1, B), jnp.float32)
    for zi in range(Z):
        nl = jnp.sum(tT * neg_ref[:, zi * B:(zi + 1) * B], axis=0,
                     keepdims=True)
        p_neg = jnp.clip(1.0 / (1.0 + jnp.exp(-nl)), -0.999, 0.999)
        nsample_log = nsample_log + jnp.log(jnp.clip(1.0 - p_neg, 0.01, None))

    out = (Z + 1) / (2 * Z) * (log_p + nsample_log)
    reg = jnp.sum(regp_ref[...]) + jnp.sum(proj_ref[...] * proj_ref[...])
    loss = jnp.sum(out) / B + 0.01 / (2 * B) * reg
    out_ref[...] = loss[None, None]


_tc_finish = pl.pallas_call(
    _tc_finish_body,
    out_shape=jax.ShapeDtypeStruct((1, 1), jnp.float32),
)


def kernel(rv, rd, proj, beta, word_ids, doc_ids, nsample_ids):
    widx = word_ids.astype(jnp.int32).T.reshape(-1)
    dnidx = jnp.concatenate(
        [doc_ids.astype(jnp.int32),
         nsample_ids.astype(jnp.int32).T.reshape(-1)])
    wp, docs, neg, regp = _sc_gather_built()(rv, rd, widx, dnidx)
    loss = _tc_finish(wp, docs, neg, regp, proj, beta)
    return loss.reshape(())


# speedup vs baseline: 1.0002x; 1.0002x over previous
import functools

import jax
import jax.numpy as jnp
from jax import lax
from jax.experimental import pallas as pl
from jax.experimental.pallas import tpu as pltpu
from jax.experimental.pallas import tpu_sc as plsc

B = 1024
NGRAM = 10
Z = 10
D = 128
V = 100000
LANES = 16
NW = 32
ROWS_PER_W = D // NW
NCHUNK = B // LANES
DN = B + Z * B
SQ_ITERS = V // LANES

@functools.cache
def _sc_gather_built():
    mesh = plsc.VectorSubcoreMesh(core_axis_name="c", subcore_axis_name="s")
    return functools.partial(
        pl.kernel,
        out_type=[
            jax.ShapeDtypeStruct((D, B), jnp.float32),
            jax.ShapeDtypeStruct((D, B), jnp.float32),
            jax.ShapeDtypeStruct((D, Z * B), jnp.float32),
            jax.ShapeDtypeStruct((NW, 128), jnp.float32),
        ],
        mesh=mesh,
        compiler_params=pltpu.CompilerParams(needs_layout_passes=False,
                                             use_tc_tiling_on_sc=True),
        scratch_types=[
            pltpu.VMEM((V,), jnp.float32),
            pltpu.VMEM((DN,), jnp.int32),
            pltpu.VMEM((DN,), jnp.float32),
        ],
    )(_sc_gather_body)


def _sc_gather_body(rv_hbm, rd_hbm, widx_hbm, dnidx_hbm,
                    wp_hbm, docs_hbm, neg_hbm, reg_hbm,
                    row_v, idx_v, out_v):
    wid = lax.axis_index("s") * 2 + lax.axis_index("c")

    pltpu.sync_copy(widx_hbm, idx_v.at[pl.ds(0, NGRAM * B)])

    def rv_row(r, carry):
        d = wid * ROWS_PER_W + r
        pltpu.sync_copy(rv_hbm.at[d], row_v)

        def chunk(c, carry2):
            acc = jnp.zeros((LANES,), jnp.float32)
            for zi in range(NGRAM):
                iv = idx_v[pl.ds(zi * B + c * LANES, LANES)]
                acc = acc + plsc.load_gather(row_v, [iv])
            out_v[pl.ds(c * LANES, LANES)] = acc
            return carry2

        lax.fori_loop(0, NCHUNK, chunk, 0)
        pltpu.sync_copy(out_v.at[pl.ds(0, B)], wp_hbm.at[d])
        return carry

    lax.fori_loop(0, ROWS_PER_W, rv_row, 0)

    pltpu.sync_copy(dnidx_hbm, idx_v)

    def rd_row(r, regacc):
        d = wid * ROWS_PER_W + r
        pltpu.sync_copy(rd_hbm.at[d], row_v)

        def sq(i, a):
            v = row_v[pl.ds(i * LANES, LANES)]
            return a + v * v

        regacc = lax.fori_loop(0, SQ_ITERS, sq, regacc)

        def chunk(c, carry2):
            iv = idx_v[pl.ds(c * LANES, LANES)]
            out_v[pl.ds(c * LANES, LANES)] = plsc.load_gather(row_v, [iv])
            return carry2

        lax.fori_loop(0, DN // LANES, chunk, 0)
        pltpu.sync_copy(out_v.at[pl.ds(0, B)], docs_hbm.at[d])
        pltpu.sync_copy(out_v.at[pl.ds(B, Z * B)], neg_hbm.at[d])
        return regacc

    regacc = lax.fori_loop(0, ROWS_PER_W, rd_row, jnp.zeros((LANES,), jnp.float32))

    out_v[pl.ds(0, LANES)] = regacc
    for j in range(1, 128 // LANES):
        out_v[pl.ds(j * LANES, LANES)] = jnp.zeros((LANES,), jnp.float32)
    pltpu.sync_copy(out_v.at[pl.ds(0, 128)], reg_hbm.at[wid])


def _tc_finish_body(wp_ref, docs_ref, neg_ref, regp_ref, proj_ref, beta_ref,
                    out_ref):
    wpT = wp_ref[...] * (1.0 / NGRAM)
    norm = jnp.sqrt(jnp.sum(wpT * wpT, axis=0, keepdims=True))
    normedT = wpT / norm
    tpreT = jnp.dot(proj_ref[...], normedT,
                    preferred_element_type=jnp.float32)
    mean = jnp.mean(tpreT, axis=1, keepdims=True)
    std = jnp.sqrt(jnp.sum((tpreT - mean) ** 2, axis=1, keepdims=True)
                   / (B - 1))
    tT = jnp.clip((tpreT - mean) / jnp.sqrt(std) + beta_ref[...], -1.0, 1.0)

    pos_logit = jnp.sum(tT * docs_ref[...], axis=0, keepdims=True)
    p_pos = jnp.clip(1.0 / (1.0 + jnp.exp(-pos_logit)), -0.999, 0.999)
    log_p = float(Z) * jnp.log(p_pos)

    nsample_log = jnp.zeros((1, B), jnp.float32)
    for zi in range(Z):
        nl = jnp.sum(tT * neg_ref[:, zi * B:(zi + 1) * B], axis=0,
                     keepdims=True)
        p_neg = jnp.clip(1.0 / (1.0 + jnp.exp(-nl)), -0.999, 0.999)
        nsample_log = nsample_log + jnp.log(jnp.clip(1.0 - p_neg, 0.01, None))

    out = (Z + 1) / (2 * Z) * (log_p + nsample_log)
    reg = jnp.sum(regp_ref[...]) + jnp.sum(proj_ref[...] * proj_ref[...])
    loss = jnp.sum(out) / B + 0.01 / (2 * B) * reg
    out_ref[...] = loss[None, None]


_tc_finish = pl.pallas_call(
    _tc_finish_body,
    out_shape=jax.ShapeDtypeStruct((1, 1), jnp.float32),
)


def kernel(rv, rd, proj, beta, word_ids, doc_ids, nsample_ids):
    widx = word_ids.astype(jnp.int32).T.reshape(-1)
    dnidx = jnp.concatenate(
        [doc_ids.astype(jnp.int32),
         nsample_ids.astype(jnp.int32).T.reshape(-1)])
    wp, docs, neg, regp = _sc_gather_built()(rv, rd, widx, dnidx)
    loss = _tc_finish(wp, docs, neg, regp, proj, beta)
    return loss.reshape(())


# speedup vs baseline: 4.7182x; 4.7174x over previous
import functools

import jax
import jax.numpy as jnp
from jax import lax
from jax.experimental import pallas as pl
from jax.experimental.pallas import tpu as pltpu
from jax.experimental.pallas import tpu_sc as plsc

B = 1024
NGRAM = 10
Z = 10
D = 128
V = 100000
LANES = 16
NW = 32
SPW = B // NW
WPW = SPW * NGRAM
NPW = SPW * Z
GCHUNK = 64


@functools.cache
def _sc_gather_built():
    mesh = plsc.VectorSubcoreMesh(core_axis_name="c", subcore_axis_name="s")
    return functools.partial(
        pl.kernel,
        out_type=[
            jax.ShapeDtypeStruct((B, D), jnp.float32),
            jax.ShapeDtypeStruct((B, D), jnp.float32),
            jax.ShapeDtypeStruct((Z * B, D), jnp.float32),
        ],
        mesh=mesh,
        compiler_params=pltpu.CompilerParams(needs_layout_passes=False),
        scratch_types=[
            pltpu.VMEM((WPW, D), jnp.float32),
            pltpu.VMEM((NPW, D), jnp.float32),
            pltpu.VMEM((SPW, D), jnp.float32),
            pltpu.VMEM((SPW, D), jnp.float32),
            pltpu.VMEM((WPW,), jnp.int32),
            pltpu.VMEM((NPW,), jnp.int32),
            pltpu.VMEM((SPW,), jnp.int32),
            pltpu.SemaphoreType.DMA,
            pltpu.SemaphoreType.DMA,
            pltpu.SemaphoreType.DMA,
        ],
    )(_sc_gather_body)


def _sc_gather_body(rvt_hbm, rdt_hbm, widx_hbm, didx_hbm, nidx_hbm,
                    wp_hbm, docs_hbm, neg_hbm,
                    wrows_v, nrows_v, drows_v, wp_v,
                    widx_v, nidx_v, didx_v, sem_w, sem_n, sem_d):
    wid = lax.axis_index("s") * 2 + lax.axis_index("c")

    pltpu.sync_copy(widx_hbm.at[pl.ds(wid * WPW, WPW)], widx_v)
    for zi in range(Z):
        pltpu.sync_copy(nidx_hbm.at[pl.ds(zi * B + wid * SPW, SPW)],
                        nidx_v.at[pl.ds(zi * SPW, SPW)])
    pltpu.sync_copy(didx_hbm.at[pl.ds(wid * SPW, SPW)], didx_v)

    copies = []
    for k in range(WPW // GCHUNK):
        copies.append(pltpu.async_copy(
            rvt_hbm.at[widx_v.at[pl.ds(k * GCHUNK, GCHUNK)]],
            wrows_v.at[pl.ds(k * GCHUNK, GCHUNK)], sem_w))
    for k in range(NPW // GCHUNK):
        copies.append(pltpu.async_copy(
            rdt_hbm.at[nidx_v.at[pl.ds(k * GCHUNK, GCHUNK)]],
            nrows_v.at[pl.ds(k * GCHUNK, GCHUNK)], sem_n))
    copies.append(pltpu.async_copy(rdt_hbm.at[didx_v], drows_v, sem_d))
    for c in copies:
        c.wait()

    pltpu.sync_copy(drows_v, docs_hbm.at[pl.ds(wid * SPW, SPW)])

    def sum_chunk(i, carry):
        s = i // (D // LANES)
        c = i % (D // LANES)
        acc = jnp.zeros((LANES,), jnp.float32)
        for zi in range(NGRAM):
            acc = acc + wrows_v[s * NGRAM + zi, pl.ds(c * LANES, LANES)]
        wp_v[s, pl.ds(c * LANES, LANES)] = acc
        return carry

    lax.fori_loop(0, SPW * (D // LANES), sum_chunk, 0)
    pltpu.sync_copy(wp_v, wp_hbm.at[pl.ds(wid * SPW, SPW)])

    for zi in range(Z):
        pltpu.sync_copy(nrows_v.at[pl.ds(zi * SPW, SPW)],
                        neg_hbm.at[pl.ds(zi * B + wid * SPW, SPW)])


def _reg_body(rdt_ref, out_ref):
    x = rdt_ref[...]
    psum = jnp.sum(x * x)

    @pl.when(pl.program_id(0) == 0)
    def _init():
        out_ref[...] = jnp.zeros((1, 1), jnp.float32)

    out_ref[...] += psum[None, None]


_REG_BLOCK = 5000
_reg_sum = pl.pallas_call(
    _reg_body,
    grid=(V // _REG_BLOCK,),
    in_specs=[pl.BlockSpec((_REG_BLOCK, D), lambda i: (i, 0))],
    out_specs=pl.BlockSpec((1, 1), lambda i: (0, 0)),
    out_shape=jax.ShapeDtypeStruct((1, 1), jnp.float32),
)


def _tc_finish_body(wp_ref, docs_ref, neg_ref, reg_ref, proj_ref, beta_ref,
                    out_ref):
    wp = wp_ref[...] * (1.0 / NGRAM)
    norm = jnp.sqrt(jnp.sum(wp * wp, axis=1, keepdims=True))
    normed = wp / norm
    t_tensor = lax.dot_general(normed, proj_ref[...],
                               (((1,), (1,)), ((), ())),
                               preferred_element_type=jnp.float32)
    mean = jnp.mean(t_tensor, axis=0, keepdims=True)
    std = jnp.sqrt(jnp.sum((t_tensor - mean) ** 2, axis=0, keepdims=True)
                   / (B - 1))
    t = jnp.clip((t_tensor - mean) / jnp.sqrt(std) + beta_ref[...], -1.0, 1.0)

    pos_logit = jnp.sum(t * docs_ref[...], axis=1, keepdims=True)
    p_pos = jnp.clip(1.0 / (1.0 + jnp.exp(-pos_logit)), -0.999, 0.999)
    acc = float(Z) * jnp.log(p_pos)
    for zi in range(Z):
        nl = jnp.sum(t * neg_ref[zi * B:(zi + 1) * B, :], axis=1,
                     keepdims=True)
        p_neg = jnp.clip(1.0 / (1.0 + jnp.exp(-nl)), -0.999, 0.999)
        acc = acc + jnp.log(jnp.clip(1.0 - p_neg, 0.01, None))

    reg = reg_ref[0, 0] + jnp.sum(proj_ref[...] * proj_ref[...])
    loss = ((Z + 1) / (2 * Z)) * jnp.sum(acc) / B + 0.01 / (2 * B) * reg
    out_ref[...] = loss[None, None]


_tc_finish = pl.pallas_call(
    _tc_finish_body,
    out_shape=jax.ShapeDtypeStruct((1, 1), jnp.float32),
)


def kernel(rv, rd, proj, beta, word_ids, doc_ids, nsample_ids):
    rvt = rv.T
    rdt = rd.T
    widx = word_ids.astype(jnp.int32).reshape(-1)
    didx = doc_ids.astype(jnp.int32)
    nidx = nsample_ids.astype(jnp.int32).T.reshape(-1)
    wp, docs, neg = _sc_gather_built()(rvt, rdt, widx, didx, nidx)
    reg = _reg_sum(rdt)
    loss = _tc_finish(wp, docs, neg, reg, proj, beta.reshape(1, D))
    return loss.reshape(())


# speedup vs baseline: 5.1901x; 1.1000x over previous
import functools

import jax
import jax.numpy as jnp
from jax import lax
from jax.experimental import pallas as pl
from jax.experimental.pallas import tpu as pltpu
from jax.experimental.pallas import tpu_sc as plsc

B = 1024
NGRAM = 10
Z = 10
D = 128
V = 100000
LANES = 16
NW = 32
SPW = B // NW
WPW = SPW * NGRAM
NPW = SPW * Z
GCHUNK = 80


@functools.cache
def _sc_gather_built():
    mesh = plsc.VectorSubcoreMesh(core_axis_name="c", subcore_axis_name="s")
    return functools.partial(
        pl.kernel,
        out_type=[
            jax.ShapeDtypeStruct((B, D), jnp.float32),
            jax.ShapeDtypeStruct((B, D), jnp.float32),
            jax.ShapeDtypeStruct((Z * B, D), jnp.float32),
        ],
        mesh=mesh,
        compiler_params=pltpu.CompilerParams(needs_layout_passes=False),
        scratch_types=[
            pltpu.VMEM((WPW, D), jnp.float32),
            pltpu.VMEM((NPW, D), jnp.float32),
            pltpu.VMEM((SPW, D), jnp.float32),
            pltpu.VMEM((SPW, D), jnp.float32),
            pltpu.VMEM((WPW,), jnp.int32),
            pltpu.VMEM((NPW,), jnp.int32),
            pltpu.VMEM((SPW,), jnp.int32),
            pltpu.SemaphoreType.DMA,
            pltpu.SemaphoreType.DMA,
            pltpu.SemaphoreType.DMA,
        ],
    )(_sc_gather_body)


def _sc_gather_body(rvt_hbm, rdt_hbm, widx_hbm, didx_hbm, nidx_hbm,
                    wp_hbm, docs_hbm, neg_hbm,
                    wrows_v, nrows_v, drows_v, wp_v,
                    widx_v, nidx_v, didx_v, sem_w, sem_n, sem_d):
    wid = lax.axis_index("s") * 2 + lax.axis_index("c")

    iw = pltpu.async_copy(widx_hbm.at[pl.ds(wid * WPW, WPW)], widx_v, sem_w)
    in_ = pltpu.async_copy(nidx_hbm.at[pl.ds(wid * NPW, NPW)], nidx_v, sem_n)
    id_ = pltpu.async_copy(didx_hbm.at[pl.ds(wid * SPW, SPW)], didx_v, sem_d)
    iw.wait()
    in_.wait()
    id_.wait()

    copies = []
    for k in range(WPW // GCHUNK):
        copies.append(pltpu.async_copy(
            rvt_hbm.at[widx_v.at[pl.ds(k * GCHUNK, GCHUNK)]],
            wrows_v.at[pl.ds(k * GCHUNK, GCHUNK)], sem_w))
    for k in range(NPW // GCHUNK):
        copies.append(pltpu.async_copy(
            rdt_hbm.at[nidx_v.at[pl.ds(k * GCHUNK, GCHUNK)]],
            nrows_v.at[pl.ds(k * GCHUNK, GCHUNK)], sem_n))
    gd = pltpu.async_copy(rdt_hbm.at[didx_v], drows_v, sem_d)

    gd.wait()
    outs = [pltpu.async_copy(drows_v, docs_hbm.at[pl.ds(wid * SPW, SPW)],
                             sem_d)]
    for c in copies:
        c.wait()
    for zi in range(Z):
        outs.append(pltpu.async_copy(
            nrows_v.at[pl.ds(zi * SPW, SPW)],
            neg_hbm.at[pl.ds(zi * B + wid * SPW, SPW)], sem_n))

    def sum_chunk(i, carry):
        s = i // (D // LANES)
        c = i % (D // LANES)
        acc = jnp.zeros((LANES,), jnp.float32)
        for zi in range(NGRAM):
            acc = acc + wrows_v[s * NGRAM + zi, pl.ds(c * LANES, LANES)]
        wp_v[s, pl.ds(c * LANES, LANES)] = acc
        return carry

    lax.fori_loop(0, SPW * (D // LANES), sum_chunk, 0)
    outs.append(pltpu.async_copy(wp_v, wp_hbm.at[pl.ds(wid * SPW, SPW)],
                                 sem_w))
    for c in outs:
        c.wait()


def _reg_body(rdt_ref, out_ref):
    x = rdt_ref[...]
    psum = jnp.sum(x * x)

    @pl.when(pl.program_id(0) == 0)
    def _init():
        out_ref[...] = jnp.zeros((1, 1), jnp.float32)

    out_ref[...] += psum[None, None]


_REG_BLOCK = 25000
_reg_sum = pl.pallas_call(
    _reg_body,
    grid=(V // _REG_BLOCK,),
    in_specs=[pl.BlockSpec((_REG_BLOCK, D), lambda i: (i, 0))],
    out_specs=pl.BlockSpec((1, 1), lambda i: (0, 0)),
    out_shape=jax.ShapeDtypeStruct((1, 1), jnp.float32),
)


def _tc_finish_body(wp_ref, docs_ref, neg_ref, reg_ref, proj_ref, beta_ref,
                    out_ref):
    wp = wp_ref[...] * (1.0 / NGRAM)
    norm = jnp.sqrt(jnp.sum(wp * wp, axis=1, keepdims=True))
    normed = wp / norm
    t_tensor = lax.dot_general(normed, proj_ref[...],
                               (((1,), (1,)), ((), ())),
                               preferred_element_type=jnp.float32)
    mean = jnp.mean(t_tensor, axis=0, keepdims=True)
    std = jnp.sqrt(jnp.sum((t_tensor - mean) ** 2, axis=0, keepdims=True)
                   / (B - 1))
    t = jnp.clip((t_tensor - mean) / jnp.sqrt(std) + beta_ref[...], -1.0, 1.0)

    pos_logit = jnp.sum(t * docs_ref[...], axis=1, keepdims=True)
    p_pos = jnp.clip(1.0 / (1.0 + jnp.exp(-pos_logit)), -0.999, 0.999)
    acc = float(Z) * jnp.log(p_pos)
    for zi in range(Z):
        nl = jnp.sum(t * neg_ref[zi * B:(zi + 1) * B, :], axis=1,
                     keepdims=True)
        p_neg = jnp.clip(1.0 / (1.0 + jnp.exp(-nl)), -0.999, 0.999)
        acc = acc + jnp.log(jnp.clip(1.0 - p_neg, 0.01, None))

    reg = reg_ref[0, 0] + jnp.sum(proj_ref[...] * proj_ref[...])
    loss = ((Z + 1) / (2 * Z)) * jnp.sum(acc) / B + 0.01 / (2 * B) * reg
    out_ref[...] = loss[None, None]


_tc_finish = pl.pallas_call(
    _tc_finish_body,
    out_shape=jax.ShapeDtypeStruct((1, 1), jnp.float32),
)


def kernel(rv, rd, proj, beta, word_ids, doc_ids, nsample_ids):
    rvt = rv.T
    rdt = rd.T
    widx = word_ids.astype(jnp.int32).reshape(-1)
    didx = doc_ids.astype(jnp.int32)
    nidx = (nsample_ids.astype(jnp.int32)
            .reshape(NW, SPW, Z).transpose(0, 2, 1).reshape(-1))
    wp, docs, neg = _sc_gather_built()(rvt, rdt, widx, didx, nidx)
    reg = _reg_sum(rdt)
    loss = _tc_finish(wp, docs, neg, reg, proj, beta.reshape(1, D))
    return loss.reshape(())
